# baseline (speedup 1.0000x reference)
import functools

import jax
import jax.numpy as jnp
from jax import lax
from jax.experimental import pallas as pl
from jax.experimental.pallas import tpu as pltpu
from jax.experimental.pallas import tpu_sc as plsc

N = 10000
E = 160000
D_IN = 256
H0 = 256
H1 = 64
D_OUT = 40

NC = 2
NS = 16
NW = NC * NS
NP = 10240
C = 128
CH = 40
C1 = 64
CH1 = 40
PH1 = 4
EP = NW * CH * C
RPS = NP // NS

_mesh = plsc.VectorSubcoreMesh(core_axis_name="c", subcore_axis_name="s")
_sc_params = pltpu.CompilerParams(needs_layout_passes=False)
_sc_params_lin = pltpu.CompilerParams(
    needs_layout_passes=False, use_tc_tiling_on_sc=False)


@functools.partial(
    pl.kernel,
    out_type=jax.ShapeDtypeStruct((NW, NP), jnp.float32),
    mesh=_mesh,
    compiler_params=_sc_params,
    scratch_types=[
        pltpu.VMEM((CH, C), jnp.int32),
        pltpu.VMEM((NP,), jnp.float32),
    ],
)
def _deg_kernel(dst_hbm, out_hbm, d_all, deg_v):
    cid = lax.axis_index("c")
    sid = lax.axis_index("s")
    wid = cid * NS + sid
    pltpu.sync_copy(dst_hbm.at[wid], d_all)

    def _zero(j, _):
        for u in range(8):
            deg_v[pl.ds((j * 8 + u) * 16, 16)] = jnp.zeros((16,), jnp.float32)
        return 0

    lax.fori_loop(0, NP // 128, _zero, 0)

    ones = jnp.ones((16,), jnp.float32)

    def _count(k, _):
        for u in range(C // 16):
            idx = d_all[k, pl.ds(u * 16, 16)]
            plsc.addupdate_scatter(deg_v, [idx], ones)
        return 0

    lax.fori_loop(0, CH, _count, 0)
    pltpu.sync_copy(deg_v, out_hbm.at[wid])


def _ring_loop(nb, lag, nchunks, g_hbm, s_all, d_all, rows, gsems, ssems,
               acc_sh):
    for b in range(nb):
        pltpu.async_copy(g_hbm.at[s_all.at[b]], rows[b], gsems[b])

    def _outer(k0, _):
        for b in range(nb):
            k = k0 * nb + b
            pltpu.make_async_copy(
                g_hbm.at[s_all.at[0]], rows[b], gsems[b]).wait()
            pltpu.async_copy(rows[b], acc_sh.at[d_all.at[k]], ssems[b],
                             add=True)

            @pl.when(k >= lag)
            def _():
                bl = (b - lag) % nb
                pltpu.make_async_copy(
                    rows[bl], acc_sh.at[d_all.at[0]], ssems[bl]).wait()
                kn = k - lag + nb

                @pl.when(kn < nchunks)
                def _():
                    pltpu.async_copy(g_hbm.at[s_all.at[kn]], rows[bl],
                                     gsems[bl])
        return 0

    lax.fori_loop(0, nchunks // nb, _outer, 0)
    for j in range(lag):
        b = (nchunks - lag + j) % nb
        pltpu.make_async_copy(
            rows[b], acc_sh.at[d_all.at[0]], ssems[b]).wait()


def _zero_acc_slice(z_hbm, acc_sh, sid):
    for j in range(RPS // C):
        pltpu.sync_copy(z_hbm, acc_sh.at[pl.ds(sid * RPS + j * C, C)])


NB1 = 4


def _edge1_body(srcb_hbm, dstb_hbm, g_hbm, z_hbm, out_hbm, s_all, d_all,
                *rest):
    rows = rest[:NB1]
    acc_sh = rest[NB1]
    gsems = rest[NB1 + 1:2 * NB1 + 1]
    ssems = rest[2 * NB1 + 1:]
    cid = lax.axis_index("c")
    sid = lax.axis_index("s")
    _zero_acc_slice(z_hbm, acc_sh, sid)
    plsc.subcore_barrier()
    for half in range(PH1):
        pltpu.sync_copy(srcb_hbm.at[cid, sid, half], s_all)
        pltpu.sync_copy(dstb_hbm.at[sid, half], d_all)
        _ring_loop(NB1, 1, CH1, g_hbm, s_all, d_all, rows, gsems, ssems,
                   acc_sh)
    plsc.subcore_barrier()
    pltpu.sync_copy(
        acc_sh.at[pl.ds(sid * RPS, RPS)],
        out_hbm.at[cid, pl.ds(sid * RPS, RPS)],
    )


_edge1 = pl.kernel(
    _edge1_body,
    out_type=jax.ShapeDtypeStruct((NC, NP, 128), jnp.float32),
    mesh=_mesh,
    compiler_params=_sc_params,
    scratch_types=[
        pltpu.VMEM((CH1, C1), jnp.int32),
        pltpu.VMEM((CH1, C1), jnp.int32),
    ] + [pltpu.VMEM((C1, 128), jnp.float32) for _ in range(NB1)] + [
        pltpu.VMEM_SHARED((NP, 128), jnp.float32),
    ] + [pltpu.SemaphoreType.DMA for _ in range(2 * NB1)],
)


NB2 = 4
CH2 = 80


def _edge32_body(srcb_hbm, dstb_hbm, g_hbm, z_hbm, out_hbm, s_all, d_all,
                 *rest):
    rows = rest[:NB2]
    acc_sh = rest[NB2]
    gsems = rest[NB2 + 1:2 * NB2 + 1]
    ssems = rest[2 * NB2 + 1:]
    cid = lax.axis_index("c")
    sid = lax.axis_index("s")
    pltpu.sync_copy(srcb_hbm.at[cid, sid], s_all)
    pltpu.sync_copy(dstb_hbm.at[sid], d_all)
    _zero_acc_slice(z_hbm, acc_sh, sid)
    plsc.subcore_barrier()
    _ring_loop(NB2, 1, CH2, g_hbm, s_all, d_all, rows, gsems, ssems, acc_sh)
    plsc.subcore_barrier()
    pltpu.sync_copy(
        acc_sh.at[pl.ds(sid * RPS, RPS)],
        out_hbm.at[cid, pl.ds(sid * RPS, RPS)],
    )


_edge32 = pl.kernel(
    _edge32_body,
    out_type=jax.ShapeDtypeStruct((NC, NP, H1 // 2), jnp.float32),
    mesh=_mesh,
    compiler_params=_sc_params_lin,
    scratch_types=[
        pltpu.VMEM((CH2, C), jnp.int32),
        pltpu.VMEM((CH2, C), jnp.int32),
    ] + [pltpu.VMEM((C, H1 // 2), jnp.float32) for _ in range(NB2)] + [
        pltpu.VMEM_SHARED((NP, H1 // 2), jnp.float32),
    ] + [pltpu.SemaphoreType.DMA for _ in range(2 * NB2)],
)


def _dis_body(deg_ref, out_ref):
    tot = jnp.sum(deg_ref[...], axis=0, keepdims=True) + 1.0
    col = lax.broadcasted_iota(jnp.int32, (1, NP), 1)
    out_ref[...] = jnp.where(col < N, lax.rsqrt(tot), 0.0)


def _dis_call(degp):
    return pl.pallas_call(
        _dis_body,
        out_shape=jax.ShapeDtypeStruct((1, NP), jnp.float32),
    )(degp)


RB = 2000
NRB = N // RB


def _mm1_body(x_ref, w_ref, out_ref):
    hw = jnp.dot(x_ref[...], w_ref[...], preferred_element_type=jnp.float32)
    out_ref[0] = hw[:, :128]
    out_ref[1] = hw[:, 128:]


def _mm1_call(x, W1):
    return pl.pallas_call(
        _mm1_body,
        grid=(NRB,),
        in_specs=[
            pl.BlockSpec((RB, D_IN), lambda i: (i, 0)),
            pl.BlockSpec((D_IN, H0), lambda i: (0, 0)),
        ],
        out_specs=pl.BlockSpec((2, RB, 128), lambda i: (0, i, 0)),
        out_shape=jax.ShapeDtypeStruct((2, NP, 128), jnp.float32),
    )(x, W1)


def _scale_body(hw_ref, dis_ref, out_ref):
    dis = dis_ref[...]
    out_ref[0] = hw_ref[0] * dis
    out_ref[1] = hw_ref[1] * dis


def _scale_call(hw, dis_col):
    return pl.pallas_call(
        _scale_body,
        grid=(NRB,),
        in_specs=[
            pl.BlockSpec((2, RB, 128), lambda i: (0, i, 0)),
            pl.BlockSpec((RB, 1), lambda i: (i, 0)),
        ],
        out_specs=pl.BlockSpec((2, RB, 128), lambda i: (0, i, 0)),
        out_shape=jax.ShapeDtypeStruct((2, NP, 128), jnp.float32),
    )(hw, dis_col)


def _mid1_body(p_ref, g_ref, dis_ref, b_ref, w_ref, out_ref):
    dis = dis_ref[...]
    h0 = jnp.maximum(dis * (p_ref[0] + g_ref[0]) + b_ref[0, :128], 0.0)
    h1 = jnp.maximum(dis * (p_ref[1] + g_ref[1]) + b_ref[0, 128:], 0.0)
    hw = (jnp.dot(h0, w_ref[:128], preferred_element_type=jnp.float32)
          + jnp.dot(h1, w_ref[128:], preferred_element_type=jnp.float32))
    g = hw * dis
    out_ref[0] = g[:, :H1 // 2]
    out_ref[1] = g[:, H1 // 2:]


def _mid1_call(p1, g1, dis_col, b1r, W2):
    return pl.pallas_call(
        _mid1_body,
        grid=(NRB,),
        in_specs=[
            pl.BlockSpec((NC, RB, 128), lambda i: (0, i, 0)),
            pl.BlockSpec((2, RB, 128), lambda i: (0, i, 0)),
            pl.BlockSpec((RB, 1), lambda i: (i, 0)),
            pl.BlockSpec((1, H0), lambda i: (0, 0)),
            pl.BlockSpec((H0, H1), lambda i: (0, 0)),
        ],
        out_specs=pl.BlockSpec((2, RB, H1 // 2), lambda i: (0, i, 0)),
        out_shape=jax.ShapeDtypeStruct((2, NP, H1 // 2), jnp.float32),
    )(p1, g1, dis_col, b1r, W2)


def _mid2_body(p_ref, g_ref, dis_ref, b_ref, w_ref, out_ref):
    dis = dis_ref[...]
    acc = jnp.concatenate([p_ref[0] + g_ref[0], p_ref[1] + g_ref[1]], axis=1)
    h = jnp.maximum(dis * acc + b_ref[...], 0.0)
    g = jnp.dot(h, w_ref[...], preferred_element_type=jnp.float32) * dis
    out_ref[0] = g[:, :H1 // 2]
    out_ref[1] = g[:, H1 // 2:]


def _mid2_call(p2, g2, dis_col, b2r, W3p):
    return pl.pallas_call(
        _mid2_body,
        grid=(NRB,),
        in_specs=[
            pl.BlockSpec((NC, RB, H1 // 2), lambda i: (0, i, 0)),
            pl.BlockSpec((2, RB, H1 // 2), lambda i: (0, i, 0)),
            pl.BlockSpec((RB, 1), lambda i: (i, 0)),
            pl.BlockSpec((1, H1), lambda i: (0, 0)),
            pl.BlockSpec((H1, H1), lambda i: (0, 0)),
        ],
        out_specs=pl.BlockSpec((2, RB, H1 // 2), lambda i: (0, i, 0)),
        out_shape=jax.ShapeDtypeStruct((2, NP, H1 // 2), jnp.float32),
    )(p2, g2, dis_col, b2r, W3p)


def _final_body(p_ref, g_ref, dis_ref, b_ref, out_ref):
    acc = jnp.concatenate([p_ref[0] + g_ref[0], p_ref[1] + g_ref[1]], axis=1)
    z = dis_ref[...] * acc + b_ref[...]
    col = lax.broadcasted_iota(jnp.int32, (RB, H1), 1)
    valid = col < D_OUT
    zm = jnp.where(valid, z, -jnp.inf)
    m = jnp.max(zm, axis=1, keepdims=True)
    e = jnp.where(valid, jnp.exp(z - m), 0.0)
    s = jnp.sum(e, axis=1, keepdims=True)
    out_ref[...] = (z - m - jnp.log(s))[:, :D_OUT]


def _final_call(p3, g3, dis_col, b3r):
    return pl.pallas_call(
        _final_body,
        grid=(NRB,),
        in_specs=[
            pl.BlockSpec((NC, RB, H1 // 2), lambda i: (0, i, 0)),
            pl.BlockSpec((2, RB, H1 // 2), lambda i: (0, i, 0)),
            pl.BlockSpec((RB, 1), lambda i: (i, 0)),
            pl.BlockSpec((1, H1), lambda i: (0, 0)),
        ],
        out_specs=pl.BlockSpec((RB, D_OUT), lambda i: (i, 0)),
        out_shape=jax.ShapeDtypeStruct((N, D_OUT), jnp.float32),
    )(p3, g3, dis_col, b3r)


def kernel(x, edge_index, W1, b1, W2, b2, W3, b3):
    src = edge_index[0]
    dst = edge_index[1]
    padi = (N + jnp.arange(EP - E, dtype=jnp.int32) % (NP - N))
    srcp = jnp.concatenate([src, padi])
    dstp = jnp.concatenate([dst, padi])
    src16 = srcp.reshape(NS, PH1, CH1, C1)
    srcb = jnp.stack([src16, src16 + NP])
    dstb = dstp.reshape(NS, PH1, CH1, C1)
    src32 = srcp.reshape(NW, CH, C)
    dst32 = dstp.reshape(NW, CH, C)
    src16b = srcp.reshape(NS, CH2, C)
    srcb2 = jnp.stack([src16b, src16b + NP])
    dstb2 = dstp.reshape(NS, CH2, C)
    z128 = jnp.zeros((C, 128), jnp.float32)
    z32 = jnp.zeros((C, H1 // 2), jnp.float32)
    b1r = b1.reshape(1, H0)
    b2r = b2.reshape(1, H1)
    W3p = jnp.pad(W3, ((0, 0), (0, H1 - D_OUT)))
    b3r = jnp.pad(b3, (0, H1 - D_OUT)).reshape(1, H1)

    hw1 = _mm1_call(x, W1)
    degp = _deg_kernel(dst32)
    dis_col = _dis_call(degp).reshape(NP, 1)
    g1 = _scale_call(hw1, dis_col)
    p1 = _edge1(srcb, dstb, g1.reshape(NC * NP, 128), z128)
    g2 = _mid1_call(p1, g1, dis_col, b1r, W2)
    p2 = _edge32(srcb2, dstb2, g2.reshape(NC * NP, H1 // 2), z32)
    g3 = _mid2_call(p2, g2, dis_col, b2r, W3p)
    p3 = _edge32(srcb2, dstb2, g3.reshape(NC * NP, H1 // 2), z32)
    return _final_call(p3, g3, dis_col, b3r)


# speedup vs baseline: 1.0209x; 1.0209x over previous
import functools

import jax
import jax.numpy as jnp
from jax import lax
from jax.experimental import pallas as pl
from jax.experimental.pallas import tpu as pltpu
from jax.experimental.pallas import tpu_sc as plsc

N = 10000
E = 160000
D_IN = 256
H0 = 256
H1 = 64
D_OUT = 40

NC = 2
NS = 16
NW = NC * NS
NP = 10240
C = 128
CH = 40
C1 = 64
CH1 = 40
PH1 = 4
EP = NW * CH * C
RPS = NP // NS

_mesh = plsc.VectorSubcoreMesh(core_axis_name="c", subcore_axis_name="s")
_sc_params = pltpu.CompilerParams(needs_layout_passes=False)
_sc_params_lin = pltpu.CompilerParams(
    needs_layout_passes=False, use_tc_tiling_on_sc=False)


@functools.partial(
    pl.kernel,
    out_type=jax.ShapeDtypeStruct((NW, NP), jnp.float32),
    mesh=_mesh,
    compiler_params=_sc_params,
    scratch_types=[
        pltpu.VMEM((CH, C), jnp.int32),
        pltpu.VMEM((NP,), jnp.float32),
    ],
)
def _deg_kernel(dst_hbm, out_hbm, d_all, deg_v):
    cid = lax.axis_index("c")
    sid = lax.axis_index("s")
    wid = cid * NS + sid
    pltpu.sync_copy(dst_hbm.at[wid], d_all)

    def _zero(j, _):
        for u in range(8):
            deg_v[pl.ds((j * 8 + u) * 16, 16)] = jnp.zeros((16,), jnp.float32)
        return 0

    lax.fori_loop(0, NP // 128, _zero, 0)

    ones = jnp.ones((16,), jnp.float32)

    def _count(k, _):
        for u in range(C // 16):
            idx = d_all[k, pl.ds(u * 16, 16)]
            plsc.addupdate_scatter(deg_v, [idx], ones)
        return 0

    lax.fori_loop(0, CH, _count, 0)
    pltpu.sync_copy(deg_v, out_hbm.at[wid])


def _ring_loop(nb, lag, nchunks, g_hbm, s_all, d_all, rows, gsems, ssems,
               acc_sh):
    for b in range(nb):
        pltpu.async_copy(g_hbm.at[s_all.at[b]], rows[b], gsems[b])

    def _outer(k0, _):
        for b in range(nb):
            k = k0 * nb + b
            pltpu.make_async_copy(
                g_hbm.at[s_all.at[0]], rows[b], gsems[b]).wait()
            pltpu.async_copy(rows[b], acc_sh.at[d_all.at[k]], ssems[b],
                             add=True)

            @pl.when(k >= lag)
            def _():
                bl = (b - lag) % nb
                pltpu.make_async_copy(
                    rows[bl], acc_sh.at[d_all.at[0]], ssems[bl]).wait()
                kn = k - lag + nb

                @pl.when(kn < nchunks)
                def _():
                    pltpu.async_copy(g_hbm.at[s_all.at[kn]], rows[bl],
                                     gsems[bl])
        return 0

    lax.fori_loop(0, nchunks // nb, _outer, 0)
    for j in range(lag):
        b = (nchunks - lag + j) % nb
        pltpu.make_async_copy(
            rows[b], acc_sh.at[d_all.at[0]], ssems[b]).wait()


def _zero_acc_slice(z_hbm, acc_sh, sid):
    for j in range(RPS // C):
        pltpu.sync_copy(z_hbm, acc_sh.at[pl.ds(sid * RPS + j * C, C)])


NB1 = 4


def _edge1_body(srcb_hbm, dstb_hbm, g_hbm, z_hbm, out_hbm, s_all, d_all,
                *rest):
    rows = rest[:NB1]
    acc_sh = rest[NB1]
    gsems = rest[NB1 + 1:2 * NB1 + 1]
    ssems = rest[2 * NB1 + 1:]
    cid = lax.axis_index("c")
    sid = lax.axis_index("s")
    _zero_acc_slice(z_hbm, acc_sh, sid)
    plsc.subcore_barrier()
    for half in range(PH1):
        pltpu.sync_copy(srcb_hbm.at[cid, sid, half], s_all)
        pltpu.sync_copy(dstb_hbm.at[sid, half], d_all)
        _ring_loop(NB1, 1, CH1, g_hbm, s_all, d_all, rows, gsems, ssems,
                   acc_sh)
    plsc.subcore_barrier()
    pltpu.sync_copy(
        acc_sh.at[pl.ds(sid * RPS, RPS)],
        out_hbm.at[cid, pl.ds(sid * RPS, RPS)],
    )


_edge1 = pl.kernel(
    _edge1_body,
    out_type=jax.ShapeDtypeStruct((NC, NP, 128), jnp.float32),
    mesh=_mesh,
    compiler_params=_sc_params,
    scratch_types=[
        pltpu.VMEM((CH1, C1), jnp.int32),
        pltpu.VMEM((CH1, C1), jnp.int32),
    ] + [pltpu.VMEM((C1, 128), jnp.float32) for _ in range(NB1)] + [
        pltpu.VMEM_SHARED((NP, 128), jnp.float32),
    ] + [pltpu.SemaphoreType.DMA for _ in range(2 * NB1)],
)


NB2 = 5


def _edge64_body(src_hbm, dst_hbm, g_hbm, z_hbm, out_hbm, s_all, d_all,
                 *rest):
    rows = rest[:NB2]
    acc_sh = rest[NB2]
    gsems = rest[NB2 + 1:2 * NB2 + 1]
    ssems = rest[2 * NB2 + 1:]
    cid = lax.axis_index("c")
    sid = lax.axis_index("s")
    wid = cid * NS + sid
    pltpu.sync_copy(src_hbm.at[wid], s_all)
    pltpu.sync_copy(dst_hbm.at[wid], d_all)
    _zero_acc_slice(z_hbm, acc_sh, sid)
    plsc.subcore_barrier()
    _ring_loop(NB2, 2, CH, g_hbm, s_all, d_all, rows, gsems, ssems, acc_sh)
    plsc.subcore_barrier()
    pltpu.sync_copy(
        acc_sh.at[pl.ds(sid * RPS, RPS)],
        out_hbm.at[cid, pl.ds(sid * RPS, RPS)],
    )


_edge64 = pl.kernel(
    _edge64_body,
    out_type=jax.ShapeDtypeStruct((NC, NP, H1), jnp.float32),
    mesh=_mesh,
    compiler_params=_sc_params_lin,
    scratch_types=[
        pltpu.VMEM((CH, C), jnp.int32),
        pltpu.VMEM((CH, C), jnp.int32),
    ] + [pltpu.VMEM((C, H1), jnp.float32) for _ in range(NB2)] + [
        pltpu.VMEM_SHARED((NP, H1), jnp.float32),
    ] + [pltpu.SemaphoreType.DMA for _ in range(2 * NB2)],
)


def _dis_body(deg_ref, out_ref):
    tot = jnp.sum(deg_ref[...], axis=0, keepdims=True) + 1.0
    col = lax.broadcasted_iota(jnp.int32, (1, NP), 1)
    out_ref[...] = jnp.where(col < N, lax.rsqrt(tot), 0.0)


def _dis_call(degp):
    return pl.pallas_call(
        _dis_body,
        out_shape=jax.ShapeDtypeStruct((1, NP), jnp.float32),
    )(degp)


RB = 2000
NRB = N // RB


def _mm1_body(x_ref, w_ref, out_ref):
    hw = jnp.dot(x_ref[...], w_ref[...], preferred_element_type=jnp.float32)
    out_ref[0] = hw[:, :128]
    out_ref[1] = hw[:, 128:]


def _mm1_call(x, W1):
    return pl.pallas_call(
        _mm1_body,
        grid=(NRB,),
        in_specs=[
            pl.BlockSpec((RB, D_IN), lambda i: (i, 0)),
            pl.BlockSpec((D_IN, H0), lambda i: (0, 0)),
        ],
        out_specs=pl.BlockSpec((2, RB, 128), lambda i: (0, i, 0)),
        out_shape=jax.ShapeDtypeStruct((2, NP, 128), jnp.float32),
    )(x, W1)


def _scale_body(hw_ref, dis_ref, out_ref):
    dis = dis_ref[...]
    out_ref[0] = hw_ref[0] * dis
    out_ref[1] = hw_ref[1] * dis


def _scale_call(hw, dis_col):
    return pl.pallas_call(
        _scale_body,
        grid=(NRB,),
        in_specs=[
            pl.BlockSpec((2, RB, 128), lambda i: (0, i, 0)),
            pl.BlockSpec((RB, 1), lambda i: (i, 0)),
        ],
        out_specs=pl.BlockSpec((2, RB, 128), lambda i: (0, i, 0)),
        out_shape=jax.ShapeDtypeStruct((2, NP, 128), jnp.float32),
    )(hw, dis_col)


def _mid1_body(p_ref, g_ref, dis_ref, b_ref, w_ref, out_ref):
    dis = dis_ref[...]
    h0 = jnp.maximum(dis * (p_ref[0] + g_ref[0]) + b_ref[0, :128], 0.0)
    h1 = jnp.maximum(dis * (p_ref[1] + g_ref[1]) + b_ref[0, 128:], 0.0)
    hw = (jnp.dot(h0, w_ref[:128], preferred_element_type=jnp.float32)
          + jnp.dot(h1, w_ref[128:], preferred_element_type=jnp.float32))
    out_ref[...] = hw * dis


def _mid1_call(p1, g1, dis_col, b1r, W2):
    return pl.pallas_call(
        _mid1_body,
        grid=(NRB,),
        in_specs=[
            pl.BlockSpec((NC, RB, 128), lambda i: (0, i, 0)),
            pl.BlockSpec((2, RB, 128), lambda i: (0, i, 0)),
            pl.BlockSpec((RB, 1), lambda i: (i, 0)),
            pl.BlockSpec((1, H0), lambda i: (0, 0)),
            pl.BlockSpec((H0, H1), lambda i: (0, 0)),
        ],
        out_specs=pl.BlockSpec((RB, H1), lambda i: (i, 0)),
        out_shape=jax.ShapeDtypeStruct((NP, H1), jnp.float32),
    )(p1, g1, dis_col, b1r, W2)


def _mid2_body(p_ref, g_ref, dis_ref, b_ref, w_ref, out_ref):
    dis = dis_ref[...]
    h = jnp.maximum(dis * (p_ref[0] + p_ref[1] + g_ref[...]) + b_ref[...], 0.0)
    out_ref[...] = jnp.dot(h, w_ref[...], preferred_element_type=jnp.float32) * dis


def _mid2_call(p2, g2, dis_col, b2r, W3p):
    return pl.pallas_call(
        _mid2_body,
        grid=(NRB,),
        in_specs=[
            pl.BlockSpec((NC, RB, H1), lambda i: (0, i, 0)),
            pl.BlockSpec((RB, H1), lambda i: (i, 0)),
            pl.BlockSpec((RB, 1), lambda i: (i, 0)),
            pl.BlockSpec((1, H1), lambda i: (0, 0)),
            pl.BlockSpec((H1, H1), lambda i: (0, 0)),
        ],
        out_specs=pl.BlockSpec((RB, H1), lambda i: (i, 0)),
        out_shape=jax.ShapeDtypeStruct((NP, H1), jnp.float32),
    )(p2, g2, dis_col, b2r, W3p)


def _final_body(p_ref, g_ref, dis_ref, b_ref, out_ref):
    z = dis_ref[...] * (p_ref[0] + p_ref[1] + g_ref[...]) + b_ref[...]
    col = lax.broadcasted_iota(jnp.int32, (RB, H1), 1)
    valid = col < D_OUT
    zm = jnp.where(valid, z, -jnp.inf)
    m = jnp.max(zm, axis=1, keepdims=True)
    e = jnp.where(valid, jnp.exp(z - m), 0.0)
    s = jnp.sum(e, axis=1, keepdims=True)
    out_ref[...] = (z - m - jnp.log(s))[:, :D_OUT]


def _final_call(p3, g3, dis_col, b3r):
    return pl.pallas_call(
        _final_body,
        grid=(NRB,),
        in_specs=[
            pl.BlockSpec((NC, RB, H1), lambda i: (0, i, 0)),
            pl.BlockSpec((RB, H1), lambda i: (i, 0)),
            pl.BlockSpec((RB, 1), lambda i: (i, 0)),
            pl.BlockSpec((1, H1), lambda i: (0, 0)),
        ],
        out_specs=pl.BlockSpec((RB, D_OUT), lambda i: (i, 0)),
        out_shape=jax.ShapeDtypeStruct((N, D_OUT), jnp.float32),
    )(p3, g3, dis_col, b3r)


def kernel(x, edge_index, W1, b1, W2, b2, W3, b3):
    src = edge_index[0]
    dst = edge_index[1]
    padi = (N + jnp.arange(EP - E, dtype=jnp.int32) % (NP - N))
    srcp = jnp.concatenate([src, padi])
    dstp = jnp.concatenate([dst, padi])
    src16 = srcp.reshape(NS, PH1, CH1, C1)
    srcb = jnp.stack([src16, src16 + NP])
    dstb = dstp.reshape(NS, PH1, CH1, C1)
    src32 = srcp.reshape(NW, CH, C)
    dst32 = dstp.reshape(NW, CH, C)
    z128 = jnp.zeros((C, 128), jnp.float32)
    z64 = jnp.zeros((C, H1), jnp.float32)
    b1r = b1.reshape(1, H0)
    b2r = b2.reshape(1, H1)
    W3p = jnp.pad(W3, ((0, 0), (0, H1 - D_OUT)))
    b3r = jnp.pad(b3, (0, H1 - D_OUT)).reshape(1, H1)

    hw1 = _mm1_call(x, W1)
    degp = _deg_kernel(dst32)
    dis_col = _dis_call(degp).reshape(NP, 1)
    g1 = _scale_call(hw1, dis_col)
    p1 = _edge1(srcb, dstb, g1.reshape(NC * NP, 128), z128)
    g2 = _mid1_call(p1, g1, dis_col, b1r, W2)
    p2 = _edge64(src32, dst32, g2, z64)
    g3 = _mid2_call(p2, g2, dis_col, b2r, W3p)
    p3 = _edge64(src32, dst32, g3, z64)
    return _final_call(p3, g3, dis_col, b3r)
